# Initial kernel scaffold; baseline (speedup 1.0000x reference)
import jax
import jax.numpy as jnp
from jax.experimental import pallas as pl
from jax.experimental.pallas import tpu as pltpu

B, N, D, E, T, K = 2, 2048, 768, 8, 16, 2
NEG_INF = jnp.float32(-jnp.inf)


def _moe_kernel(ids_ref, tt_ref, wg_ref, bg_ref, x_ref, w_ref, b_ref,
                out_ref, gates_ref):
    b_id = pl.program_id(0)
    e_id = pl.program_id(1)
    x2 = x_ref[0]

    @pl.when(e_id == 0)
    def _compute_gates():
        tid = ids_ref[b_id]
        rows = jax.lax.broadcasted_iota(jnp.int32, (T, 1), 0)
        tvec = jnp.sum(jnp.where(rows == tid, tt_ref[...], 0.0), axis=0,
                       keepdims=True)
        logits = (jnp.dot(x2, wg_ref[:D, :], preferred_element_type=jnp.float32)
                  + jnp.dot(tvec, wg_ref[D:, :], preferred_element_type=jnp.float32)
                  + bg_ref[...])
        lane = jax.lax.broadcasted_iota(jnp.int32, (N, E), 1)
        m1 = jnp.max(logits, axis=1, keepdims=True)
        idx1 = jnp.min(jnp.where(logits == m1, lane, E), axis=1, keepdims=True)
        masked = jnp.where(lane == idx1, NEG_INF, logits)
        m2 = jnp.max(masked, axis=1, keepdims=True)
        idx2 = jnp.min(jnp.where(masked == m2, lane, E), axis=1, keepdims=True)
        e2 = jnp.exp(m2 - m1)
        inv_s = 1.0 / (1.0 + e2)
        p1 = inv_s
        p2 = e2 * inv_s
        gates = (jnp.where(lane == idx1, p1, 0.0)
                 + jnp.where(lane == idx2, p2, 0.0))
        omega = 1.0 - p1
        gates_ref[...] = jnp.concatenate(
            [gates, omega, jnp.zeros((N, 16 - E - 1), jnp.float32)], axis=1)

    h = jnp.dot(x2, w_ref[0], preferred_element_type=jnp.float32) + b_ref[0]
    g = jax.nn.gelu(h, approximate=False)
    lane16 = jax.lax.broadcasted_iota(jnp.int32, (N, 16), 1)
    gsel = jnp.sum(jnp.where(lane16 == e_id, gates_ref[...], 0.0), axis=1,
                   keepdims=True)
    contrib = gsel * g

    @pl.when(e_id == 0)
    def _init():
        out_ref[0] = contrib

    @pl.when(e_id != 0)
    def _acc():
        out_ref[0] = out_ref[0] + contrib


@jax.jit
def kernel(tokens, task_ids, task_table, Wg, bg, We, be, Wu, bu):
    w_all = jnp.concatenate([We, Wu[None]], axis=0)
    b_all = jnp.concatenate([be, bu[None]], axis=0)[:, None, :]
    bg2 = bg.reshape(1, E)
    ids = task_ids.astype(jnp.int32)

    grid_spec = pltpu.PrefetchScalarGridSpec(
        num_scalar_prefetch=1,
        grid=(B, E + 1),
        in_specs=[
            pl.BlockSpec((T, D), lambda b, e, ids: (0, 0)),
            pl.BlockSpec((2 * D, E), lambda b, e, ids: (0, 0)),
            pl.BlockSpec((1, E), lambda b, e, ids: (0, 0)),
            pl.BlockSpec((1, N, D), lambda b, e, ids: (b, 0, 0)),
            pl.BlockSpec((1, D, D), lambda b, e, ids: (e, 0, 0)),
            pl.BlockSpec((1, 1, D), lambda b, e, ids: (e, 0, 0)),
        ],
        out_specs=pl.BlockSpec((1, N, D), lambda b, e, ids: (b, 0, 0)),
        scratch_shapes=[pltpu.VMEM((N, 16), jnp.float32)],
    )
    return pl.pallas_call(
        _moe_kernel,
        grid_spec=grid_spec,
        out_shape=jax.ShapeDtypeStruct((B, N, D), jnp.float32),
        compiler_params=pltpu.CompilerParams(
            dimension_semantics=("arbitrary", "arbitrary"),
        ),
    )(ids, task_table, Wg, bg2, tokens, w_all, b_all)


# speedup vs baseline: 3.3295x; 3.3295x over previous
import jax
import jax.numpy as jnp
from jax.experimental import pallas as pl
from jax.experimental.pallas import tpu as pltpu

B, N, D, E, T, K = 2, 2048, 768, 8, 16, 2
NEG_INF = float("-inf")


def _moe_kernel(ids_ref, tt_ref, wg_ref, bg_ref, x_ref, w_ref, b_ref,
                out_ref, gates_ref):
    b_id = pl.program_id(0)
    e_id = pl.program_id(1)
    x2 = x_ref[0]

    @pl.when(e_id == 0)
    def _compute_gates():
        tid = ids_ref[b_id]
        rows = jax.lax.broadcasted_iota(jnp.int32, (T, 1), 0)
        tvec = jnp.sum(jnp.where(rows == tid, tt_ref[...], 0.0), axis=0,
                       keepdims=True)
        logits = (jnp.dot(x2, wg_ref[:D, :], preferred_element_type=jnp.float32)
                  + jnp.dot(tvec, wg_ref[D:, :], preferred_element_type=jnp.float32)
                  + bg_ref[...])
        lane = jax.lax.broadcasted_iota(jnp.int32, (N, E), 1)
        m1 = jnp.max(logits, axis=1, keepdims=True)
        idx1 = jnp.min(jnp.where(logits == m1, lane, E), axis=1, keepdims=True)
        masked = jnp.where(lane == idx1, NEG_INF, logits)
        m2 = jnp.max(masked, axis=1, keepdims=True)
        idx2 = jnp.min(jnp.where(masked == m2, lane, E), axis=1, keepdims=True)
        e2 = jnp.exp(m2 - m1)
        inv_s = 1.0 / (1.0 + e2)
        p1 = inv_s
        p2 = e2 * inv_s
        gates = (jnp.where(lane == idx1, p1, 0.0)
                 + jnp.where(lane == idx2, p2, 0.0))
        omega = 1.0 - p1
        gates_ref[...] = jnp.concatenate(
            [gates, omega, jnp.zeros((N, 16 - E - 1), jnp.float32)], axis=1)

    h = jnp.dot(x2, w_ref[0], preferred_element_type=jnp.float32) + b_ref[0]
    g = 0.5 * h * (1.0 + jax.lax.erf(h * 0.7071067811865476))
    lane16 = jax.lax.broadcasted_iota(jnp.int32, (N, 16), 1)
    gsel = jnp.sum(jnp.where(lane16 == e_id, gates_ref[...], 0.0), axis=1,
                   keepdims=True)
    contrib = gsel * g

    @pl.when(e_id == 0)
    def _init():
        out_ref[0] = contrib

    @pl.when(e_id != 0)
    def _acc():
        out_ref[0] = out_ref[0] + contrib


@jax.jit
def kernel(tokens, task_ids, task_table, Wg, bg, We, be, Wu, bu):
    w_all = jnp.concatenate([We, Wu[None]], axis=0)
    b_all = jnp.concatenate([be, bu[None]], axis=0)[:, None, :]
    bg2 = bg.reshape(1, E)
    ids = task_ids.astype(jnp.int32)

    grid_spec = pltpu.PrefetchScalarGridSpec(
        num_scalar_prefetch=1,
        grid=(B, E + 1),
        in_specs=[
            pl.BlockSpec((T, D), lambda b, e, ids: (0, 0)),
            pl.BlockSpec((2 * D, E), lambda b, e, ids: (0, 0)),
            pl.BlockSpec((1, E), lambda b, e, ids: (0, 0)),
            pl.BlockSpec((1, N, D), lambda b, e, ids: (b, 0, 0)),
            pl.BlockSpec((1, D, D), lambda b, e, ids: (e, 0, 0)),
            pl.BlockSpec((1, 1, D), lambda b, e, ids: (e, 0, 0)),
        ],
        out_specs=pl.BlockSpec((1, N, D), lambda b, e, ids: (b, 0, 0)),
        scratch_shapes=[pltpu.VMEM((N, 16), jnp.float32)],
    )
    return pl.pallas_call(
        _moe_kernel,
        grid_spec=grid_spec,
        out_shape=jax.ShapeDtypeStruct((B, N, D), jnp.float32),
        compiler_params=pltpu.CompilerParams(
            dimension_semantics=("arbitrary", "arbitrary"),
        ),
    )(ids, task_table, Wg, bg2, tokens, w_all, b_all)


# speedup vs baseline: 3.4080x; 1.0236x over previous
import jax
import jax.numpy as jnp
from jax.experimental import pallas as pl
from jax.experimental.pallas import tpu as pltpu

B, N, D, E, T, K = 2, 2048, 768, 8, 16, 2
NEG_INF = float("-inf")


def _moe_kernel(ids_ref, tt_ref, wg_ref, bg_ref, x_ref, w_ref, b_ref,
                out_ref, gates_ref, xbf_ref):
    b_id = pl.program_id(0)
    e_id = pl.program_id(1)
    x2 = x_ref[0]

    @pl.when(e_id == 0)
    def _compute_gates():
        xbf_ref[...] = x2.astype(jnp.bfloat16)
        tid = ids_ref[b_id]
        rows = jax.lax.broadcasted_iota(jnp.int32, (T, 1), 0)
        tvec = jnp.sum(jnp.where(rows == tid, tt_ref[...], 0.0), axis=0,
                       keepdims=True)
        logits = (jnp.dot(x2, wg_ref[:D, :], preferred_element_type=jnp.float32)
                  + jnp.dot(tvec, wg_ref[D:, :], preferred_element_type=jnp.float32)
                  + bg_ref[...])
        lane = jax.lax.broadcasted_iota(jnp.int32, (N, E), 1)
        m1 = jnp.max(logits, axis=1, keepdims=True)
        idx1 = jnp.min(jnp.where(logits == m1, lane, E), axis=1, keepdims=True)
        masked = jnp.where(lane == idx1, NEG_INF, logits)
        m2 = jnp.max(masked, axis=1, keepdims=True)
        idx2 = jnp.min(jnp.where(masked == m2, lane, E), axis=1, keepdims=True)
        e2 = jnp.exp(m2 - m1)
        inv_s = 1.0 / (1.0 + e2)
        p1 = inv_s
        p2 = e2 * inv_s
        gates = (jnp.where(lane == idx1, p1, 0.0)
                 + jnp.where(lane == idx2, p2, 0.0))
        omega = 1.0 - p1
        gates_ref[...] = jnp.concatenate(
            [gates, omega, jnp.zeros((N, 16 - E - 1), jnp.float32)], axis=1)

    h = jnp.dot(xbf_ref[...], w_ref[0],
                preferred_element_type=jnp.float32) + b_ref[0]
    g = 0.5 * h * (1.0 + jax.lax.erf(h * 0.7071067811865476))
    lane16 = jax.lax.broadcasted_iota(jnp.int32, (N, 16), 1)
    gsel = jnp.sum(jnp.where(lane16 == e_id, gates_ref[...], 0.0), axis=1,
                   keepdims=True)
    contrib = gsel * g

    @pl.when(e_id == 0)
    def _init():
        out_ref[0] = contrib

    @pl.when(e_id != 0)
    def _acc():
        out_ref[0] = out_ref[0] + contrib


@jax.jit
def kernel(tokens, task_ids, task_table, Wg, bg, We, be, Wu, bu):
    w_all = jnp.concatenate([We, Wu[None]], axis=0).astype(jnp.bfloat16)
    b_all = jnp.concatenate([be, bu[None]], axis=0)[:, None, :]
    bg2 = bg.reshape(1, E)
    ids = task_ids.astype(jnp.int32)

    grid_spec = pltpu.PrefetchScalarGridSpec(
        num_scalar_prefetch=1,
        grid=(B, E + 1),
        in_specs=[
            pl.BlockSpec((T, D), lambda b, e, ids: (0, 0)),
            pl.BlockSpec((2 * D, E), lambda b, e, ids: (0, 0)),
            pl.BlockSpec((1, E), lambda b, e, ids: (0, 0)),
            pl.BlockSpec((1, N, D), lambda b, e, ids: (b, 0, 0)),
            pl.BlockSpec((1, D, D), lambda b, e, ids: (e, 0, 0)),
            pl.BlockSpec((1, 1, D), lambda b, e, ids: (e, 0, 0)),
        ],
        out_specs=pl.BlockSpec((1, N, D), lambda b, e, ids: (b, 0, 0)),
        scratch_shapes=[pltpu.VMEM((N, 16), jnp.float32),
                        pltpu.VMEM((N, D), jnp.bfloat16)],
    )
    return pl.pallas_call(
        _moe_kernel,
        grid_spec=grid_spec,
        out_shape=jax.ShapeDtypeStruct((B, N, D), jnp.float32),
        compiler_params=pltpu.CompilerParams(
            dimension_semantics=("arbitrary", "arbitrary"),
        ),
    )(ids, task_table, Wg, bg2, tokens, w_all, b_all)


# speedup vs baseline: 4.0157x; 1.1783x over previous
import jax
import jax.numpy as jnp
from jax.experimental import pallas as pl
from jax.experimental.pallas import tpu as pltpu

B, N, D, E, T, K = 2, 2048, 768, 8, 16, 2
NE = E + 1
BLK = 512
NEG_INF = float("-inf")


def _moe_kernel(ids_ref, tt_ref, wg_ref, bg_ref, x_ref, w_ref, b_ref, out_ref):
    s_id = pl.program_id(0)
    xb = x_ref[...]

    tid = ids_ref[s_id // (N // BLK)]
    rows = jax.lax.broadcasted_iota(jnp.int32, (T, 1), 0)
    tvec = jnp.sum(jnp.where(rows == tid, tt_ref[...], 0.0), axis=0,
                   keepdims=True).astype(jnp.bfloat16)
    logits = (jnp.dot(xb, wg_ref[:D, :], preferred_element_type=jnp.float32)
              + jnp.dot(tvec, wg_ref[D:, :], preferred_element_type=jnp.float32)
              + bg_ref[...])
    lane = jax.lax.broadcasted_iota(jnp.int32, (BLK, E), 1)
    m1 = jnp.max(logits, axis=1, keepdims=True)
    idx1 = jnp.min(jnp.where(logits == m1, lane, E), axis=1, keepdims=True)
    masked = jnp.where(lane == idx1, NEG_INF, logits)
    m2 = jnp.max(masked, axis=1, keepdims=True)
    idx2 = jnp.min(jnp.where(masked == m2, lane, E), axis=1, keepdims=True)
    e2 = jnp.exp(m2 - m1)
    inv_s = 1.0 / (1.0 + e2)
    p1 = inv_s
    p2 = e2 * inv_s
    gates = (jnp.where(lane == idx1, p1, 0.0)
             + jnp.where(lane == idx2, p2, 0.0))
    omega = 1.0 - p1

    acc = jnp.zeros((BLK, D), jnp.float32)
    for e in range(NE):
        h = jnp.dot(xb, w_ref[e], preferred_element_type=jnp.float32) + b_ref[e]
        g = 0.5 * h * (1.0 + jax.lax.erf(h * 0.7071067811865476))
        gcol = omega if e == E else gates[:, e:e + 1]
        acc = acc + gcol * g
    out_ref[...] = acc


@jax.jit
def kernel(tokens, task_ids, task_table, Wg, bg, We, be, Wu, bu):
    x = tokens.reshape(B * N, D).astype(jnp.bfloat16)
    w_all = jnp.concatenate([We, Wu[None]], axis=0).astype(jnp.bfloat16)
    b_all = jnp.concatenate([be, bu[None]], axis=0)[:, None, :]
    wg_bf = Wg.astype(jnp.bfloat16)
    bg2 = bg.reshape(1, E)
    ids = task_ids.astype(jnp.int32)

    grid_spec = pltpu.PrefetchScalarGridSpec(
        num_scalar_prefetch=1,
        grid=(B * N // BLK,),
        in_specs=[
            pl.BlockSpec((T, D), lambda s, ids: (0, 0)),
            pl.BlockSpec((2 * D, E), lambda s, ids: (0, 0)),
            pl.BlockSpec((1, E), lambda s, ids: (0, 0)),
            pl.BlockSpec((BLK, D), lambda s, ids: (s, 0)),
            pl.BlockSpec((NE, D, D), lambda s, ids: (0, 0, 0)),
            pl.BlockSpec((NE, 1, D), lambda s, ids: (0, 0, 0)),
        ],
        out_specs=pl.BlockSpec((BLK, D), lambda s, ids: (s, 0)),
    )
    out = pl.pallas_call(
        _moe_kernel,
        grid_spec=grid_spec,
        out_shape=jax.ShapeDtypeStruct((B * N, D), jnp.float32),
        compiler_params=pltpu.CompilerParams(
            dimension_semantics=("arbitrary",),
        ),
    )(ids, task_table, wg_bf, bg2, x, w_all, b_all)
    return out.reshape(B, N, D)


# speedup vs baseline: 4.1992x; 1.0457x over previous
import jax
import jax.numpy as jnp
from jax.experimental import pallas as pl
from jax.experimental.pallas import tpu as pltpu

B, N, D, E, T, K = 2, 2048, 768, 8, 16, 2
NE = E + 1
BLK = 1024
NEG_INF = float("-inf")


def _moe_kernel(ids_ref, tt_ref, wg_ref, bg_ref, x_ref, w_ref, b_ref, out_ref):
    s_id = pl.program_id(0)
    xb = x_ref[...]

    tid = ids_ref[s_id // (N // BLK)]
    rows = jax.lax.broadcasted_iota(jnp.int32, (T, 1), 0)
    tvec = jnp.sum(jnp.where(rows == tid, tt_ref[...], 0.0), axis=0,
                   keepdims=True).astype(jnp.bfloat16)
    logits = (jnp.dot(xb, wg_ref[:D, :], preferred_element_type=jnp.float32)
              + jnp.dot(tvec, wg_ref[D:, :], preferred_element_type=jnp.float32)
              + bg_ref[...])
    lane = jax.lax.broadcasted_iota(jnp.int32, (BLK, E), 1)
    m1 = jnp.max(logits, axis=1, keepdims=True)
    idx1 = jnp.min(jnp.where(logits == m1, lane, E), axis=1, keepdims=True)
    masked = jnp.where(lane == idx1, NEG_INF, logits)
    m2 = jnp.max(masked, axis=1, keepdims=True)
    idx2 = jnp.min(jnp.where(masked == m2, lane, E), axis=1, keepdims=True)
    e2 = jnp.exp(m2 - m1)
    inv_s = 1.0 / (1.0 + e2)
    p1 = inv_s
    p2 = e2 * inv_s
    gates = (jnp.where(lane == idx1, p1, 0.0)
             + jnp.where(lane == idx2, p2, 0.0))
    omega = 1.0 - p1

    acc = jnp.zeros((BLK, D), jnp.float32)
    for e in range(NE):
        h = jnp.dot(xb, w_ref[e], preferred_element_type=jnp.float32) + b_ref[e]
        g = 0.5 * h * (1.0 + jax.lax.erf(h * 0.7071067811865476))
        gcol = omega if e == E else gates[:, e:e + 1]
        acc = acc + gcol * g
    out_ref[...] = acc


@jax.jit
def kernel(tokens, task_ids, task_table, Wg, bg, We, be, Wu, bu):
    x = tokens.reshape(B * N, D).astype(jnp.bfloat16)
    w_all = jnp.concatenate([We, Wu[None]], axis=0).astype(jnp.bfloat16)
    b_all = jnp.concatenate([be, bu[None]], axis=0)[:, None, :]
    wg_bf = Wg.astype(jnp.bfloat16)
    bg2 = bg.reshape(1, E)
    ids = task_ids.astype(jnp.int32)

    grid_spec = pltpu.PrefetchScalarGridSpec(
        num_scalar_prefetch=1,
        grid=(B * N // BLK,),
        in_specs=[
            pl.BlockSpec((T, D), lambda s, ids: (0, 0)),
            pl.BlockSpec((2 * D, E), lambda s, ids: (0, 0)),
            pl.BlockSpec((1, E), lambda s, ids: (0, 0)),
            pl.BlockSpec((BLK, D), lambda s, ids: (s, 0)),
            pl.BlockSpec((NE, D, D), lambda s, ids: (0, 0, 0)),
            pl.BlockSpec((NE, 1, D), lambda s, ids: (0, 0, 0)),
        ],
        out_specs=pl.BlockSpec((BLK, D), lambda s, ids: (s, 0)),
    )
    out = pl.pallas_call(
        _moe_kernel,
        grid_spec=grid_spec,
        out_shape=jax.ShapeDtypeStruct((B * N, D), jnp.float32),
        compiler_params=pltpu.CompilerParams(
            dimension_semantics=("arbitrary",),
        ),
    )(ids, task_table, wg_bf, bg2, x, w_all, b_all)
    return out.reshape(B, N, D)
